# Initial kernel scaffold; baseline (speedup 1.0000x reference)
import jax
import jax.numpy as jnp
from jax.experimental import pallas as pl


def kernel(X, X_m, Y, Y_m, Z, Z_m, Z_t, mask, residue_idx, chain_encoding_all, pe_w, pe_b, ee_w, ln_g, ln_b):
    raise NotImplementedError("write your pallas kernel here")


# speedup vs baseline: 1.5491x; 1.5491x over previous
import functools

import jax
import jax.numpy as jnp
import numpy as np
from jax import lax
from jax.experimental import pallas as pl
from jax.experimental.pallas import tpu as pltpu
from jax.experimental.pallas import tpu_sc as plsc

_B, _L = 2, 512
_K = 30
_NUM_RBF = 16
_NUM_PE = 16
_EDGE_F = 128
_MAXREL = 32
_TW = 128
_NPAIR = 25
_RBF_TOT = _NPAIR * _NUM_RBF

_ATOM = {"N": 0, "Ca": 1, "C": 2, "O": 3, "Cb": 4}
_PAIRS = [("Ca", "Ca"),
          ("N", "N"), ("C", "C"), ("O", "O"), ("Cb", "Cb"), ("Ca", "N"),
          ("Ca", "C"), ("Ca", "O"), ("Ca", "Cb"), ("N", "C"), ("N", "O"),
          ("N", "Cb"), ("Cb", "C"), ("Cb", "O"), ("O", "C"), ("N", "Ca"),
          ("C", "Ca"), ("O", "Ca"), ("Cb", "Ca"), ("C", "N"), ("O", "N"),
          ("Cb", "N"), ("C", "Cb"), ("O", "Cb"), ("C", "O")]


def _build_consts():
    sel = np.zeros((2 * _TW, 3 * _NPAIR), dtype=np.float32)
    for p, (ai, aj) in enumerate(_PAIRS):
        for c in range(3):
            sel[3 * _ATOM[ai] + c, 3 * p + c] = 1.0
            sel[_TW + 3 * _ATOM[aj] + c, 3 * p + c] = -1.0
    t = np.zeros((3 * _NPAIR, _RBF_TOT), dtype=np.float32)
    for p in range(_NPAIR):
        for c in range(3):
            for m in range(_NUM_RBF):
                t[3 * p + c, p * _NUM_RBF + m] = 1.0
    mu = np.linspace(2.0, 22.0, _NUM_RBF).astype(np.float32)
    mu = np.tile(mu, _NPAIR).reshape(1, _RBF_TOT)
    return jnp.asarray(sel), jnp.asarray(t), jnp.asarray(mu)


def _topk_body(xr_ref, xt_ref, rid_ref, ch_ref, tab_ref, eidx_ref, d_ref):
    xr = xr_ref[0]
    n_a = xr[:, 0:3]
    ca = xr[:, 3:6]
    c_a = xr[:, 6:9]
    b_v = ca - n_a
    c_v = c_a - ca
    bx, by, bz = b_v[:, 0:1], b_v[:, 1:2], b_v[:, 2:3]
    cx, cy, cz = c_v[:, 0:1], c_v[:, 1:2], c_v[:, 2:3]
    a_v = jnp.concatenate(
        [by * cz - bz * cy, bz * cx - bx * cz, bx * cy - by * cx], axis=1)
    cb = -0.58273431 * a_v + 0.56802827 * b_v - 0.54067466 * c_v + ca

    rid = rid_ref[0]
    ch = ch_ref[0]
    pad = jnp.zeros((_L, _TW - 17), dtype=jnp.float32)
    tab_ref[0] = jnp.concatenate(
        [xr[:, 0:3], ca, xr[:, 6:12], cb, rid, ch, pad], axis=1)

    xt = xt_ref[0]
    d2 = jnp.zeros((_L, _L), dtype=jnp.float32)
    for c in range(3):
        dx = ca[:, c:c + 1] - xt[3 + c:4 + c, :]
        d2 = d2 + dx * dx
    d_ref[...] = jnp.sqrt(d2 + 1e-6)

    lane = lax.broadcasted_iota(jnp.int32, (_L, _L), 1)
    big = jnp.int32(1 << 20)
    for k in range(_K):
        d = d_ref[...]
        m = jnp.min(d, axis=1, keepdims=True)
        am = jnp.min(jnp.where(d == m, lane, big), axis=1, keepdims=True)
        eidx_ref[0, :, k:k + 1] = am
        d_ref[...] = jnp.where(lane == am, jnp.float32(jnp.inf), d)


def _stage_a(xr, xt, ridf, chf):
    return pl.pallas_call(
        _topk_body,
        grid=(_B,),
        in_specs=[
            pl.BlockSpec((1, _L, 12), lambda b: (b, 0, 0)),
            pl.BlockSpec((1, 12, _L), lambda b: (b, 0, 0)),
            pl.BlockSpec((1, _L, 1), lambda b: (b, 0, 0)),
            pl.BlockSpec((1, _L, 1), lambda b: (b, 0, 0)),
        ],
        out_specs=[
            pl.BlockSpec((1, _L, _TW), lambda b: (b, 0, 0)),
            pl.BlockSpec((1, _L, _K), lambda b: (b, 0, 0)),
        ],
        out_shape=[
            jax.ShapeDtypeStruct((_B, _L, _TW), jnp.float32),
            jax.ShapeDtypeStruct((_B, _L, _K), jnp.int32),
        ],
        scratch_shapes=[pltpu.VMEM((_L, _L), jnp.float32)],
    )(xr, xt, ridf, chf)


_NE = _B * _L * _K
_IDX_CH = 120
_N_CH = 8


def _sc_gather(tab_flat, gidx_j):
    info = plsc.get_sparse_core_info()
    nw = info.num_cores * info.num_subcores
    per_w = _NE // nw
    assert per_w == _IDX_CH * _N_CH

    mesh = plsc.VectorSubcoreMesh(core_axis_name="c", subcore_axis_name="s")

    @functools.partial(
        pl.kernel,
        out_type=jax.ShapeDtypeStruct((_NE, _TW), jnp.float32),
        mesh=mesh,
        scratch_types=[
            pltpu.VMEM((per_w,), jnp.int32),
            pltpu.VMEM((per_w, _TW), jnp.float32),
            pltpu.SemaphoreType.DMA,
        ],
    )
    def gather_k(tab_hbm, gj_hbm, oj_hbm, ij_v, rj_v, sem):
        wid = lax.axis_index("s") * info.num_cores + lax.axis_index("c")
        base = wid * per_w
        pltpu.sync_copy(gj_hbm.at[pl.ds(base, per_w)], ij_v)
        for c in range(_N_CH):
            sl = pl.ds(c * _IDX_CH, _IDX_CH)
            pltpu.async_copy(tab_hbm.at[ij_v.at[sl]], rj_v.at[sl], sem).wait()
        pltpu.sync_copy(rj_v, oj_hbm.at[pl.ds(base, per_w)])

    return gather_k(tab_flat, gidx_j)


_BLK = 960
_RPB = _BLK // _K


def _feat_body(tab_ref, gj_ref, sel_ref, t_ref, mu_ref, pew_ref, peb_ref,
               eew_ref, lng_ref, lnb_ref, out_ref):
    row_of_edge = lax.broadcasted_iota(jnp.int32, (_BLK, _RPB), 0) // _K
    col = lax.broadcasted_iota(jnp.int32, (_BLK, _RPB), 1)
    rep = (row_of_edge == col).astype(jnp.float32)
    gi = jnp.dot(rep, tab_ref[...],
                 preferred_element_type=jnp.float32,
                 precision=lax.Precision.HIGHEST)
    gj = gj_ref[...]
    gij = jnp.concatenate([gi, gj], axis=1)
    dif = jnp.dot(gij, sel_ref[...],
                  preferred_element_type=jnp.float32,
                 precision=lax.Precision.HIGHEST)
    d2 = jnp.dot(dif * dif, t_ref[...],
                 preferred_element_type=jnp.float32,
                 precision=lax.Precision.HIGHEST)
    dist = jnp.sqrt(d2 + 1e-6)
    z = (dist - mu_ref[...]) * jnp.float32(_NUM_RBF / 20.0)
    rbf = jnp.exp(-(z * z))

    rid_i = gi[:, 15:16]
    rid_j = gj[:, 15:16]
    same = (gi[:, 16:17] == gj[:, 16:17]).astype(jnp.float32)
    off = jnp.clip(rid_i - rid_j + _MAXREL, 0.0, 2.0 * _MAXREL)
    d_pe = off * same + (1.0 - same) * (2.0 * _MAXREL + 1.0)
    lane = lax.broadcasted_iota(jnp.int32, (_BLK, 128), 1)
    onehot = (lane == d_pe.astype(jnp.int32)).astype(jnp.float32)
    e_pos = jnp.dot(onehot, pew_ref[...],
                    preferred_element_type=jnp.float32,
                 precision=lax.Precision.HIGHEST) + peb_ref[...]

    e = (jnp.dot(e_pos, eew_ref[0:_NUM_PE, :],
                 preferred_element_type=jnp.float32,
                 precision=lax.Precision.HIGHEST)
         + jnp.dot(rbf, eew_ref[_NUM_PE:, :],
                   preferred_element_type=jnp.float32,
                 precision=lax.Precision.HIGHEST))
    mu_e = jnp.mean(e, axis=1, keepdims=True)
    ec = e - mu_e
    var = jnp.mean(ec * ec, axis=1, keepdims=True)
    out_ref[...] = ec * jax.lax.rsqrt(var + 1e-5) * lng_ref[...] \
        + lnb_ref[...]


def _stage_c(tab_flat, gj, sel, t, mu, pew_pad, peb, eew, lng, lnb):
    nprog = _NE // _BLK
    return pl.pallas_call(
        _feat_body,
        grid=(nprog,),
        in_specs=[
            pl.BlockSpec((_RPB, _TW), lambda i: (i, 0)),
            pl.BlockSpec((_BLK, _TW), lambda i: (i, 0)),
            pl.BlockSpec((2 * _TW, 3 * _NPAIR), lambda i: (0, 0)),
            pl.BlockSpec((3 * _NPAIR, _RBF_TOT), lambda i: (0, 0)),
            pl.BlockSpec((1, _RBF_TOT), lambda i: (0, 0)),
            pl.BlockSpec((128, _NUM_PE), lambda i: (0, 0)),
            pl.BlockSpec((1, _NUM_PE), lambda i: (0, 0)),
            pl.BlockSpec((_NUM_PE + _RBF_TOT, _EDGE_F), lambda i: (0, 0)),
            pl.BlockSpec((1, _EDGE_F), lambda i: (0, 0)),
            pl.BlockSpec((1, _EDGE_F), lambda i: (0, 0)),
        ],
        out_specs=pl.BlockSpec((_BLK, _EDGE_F), lambda i: (i, 0)),
        out_shape=jax.ShapeDtypeStruct((_NE, _EDGE_F), jnp.float32),
    )(tab_flat, gj, sel, t, mu, pew_pad, peb, eew, lng, lnb)


def kernel(X, X_m, Y, Y_m, Z, Z_m, Z_t, mask, residue_idx,
           chain_encoding_all, pe_w, pe_b, ee_w, ln_g, ln_b):
    xr = X.reshape(_B, _L, 12)
    xt = jnp.swapaxes(xr, 1, 2)
    ridf = residue_idx.astype(jnp.float32).reshape(_B, _L, 1)
    chf = chain_encoding_all.astype(jnp.float32).reshape(_B, _L, 1)

    table, e_idx = _stage_a(xr, xt, ridf, chf)

    gidx_j = (e_idx + (jnp.arange(_B, dtype=jnp.int32) * _L)[:, None, None]
              ).reshape(_NE)
    tab_flat = table.reshape(_B * _L, _TW)

    gj = _sc_gather(tab_flat, gidx_j)

    sel, t, mu = _build_consts()
    pew_pad = jnp.zeros((128, _NUM_PE), jnp.float32).at[0:66].set(pe_w)
    e = _stage_c(tab_flat, gj, sel, t, mu, pew_pad, pe_b.reshape(1, _NUM_PE),
                 ee_w, ln_g.reshape(1, _EDGE_F), ln_b.reshape(1, _EDGE_F))
    return e.reshape(_B, _L, _K, _EDGE_F), e_idx


# speedup vs baseline: 2.3336x; 1.5064x over previous
import functools

import jax
import jax.numpy as jnp
import numpy as np
from jax import lax
from jax.experimental import pallas as pl
from jax.experimental.pallas import tpu as pltpu
from jax.experimental.pallas import tpu_sc as plsc

_B, _L = 2, 512
_K = 30
_NUM_RBF = 16
_NUM_PE = 16
_EDGE_F = 128
_MAXREL = 32
_TW = 128
_NPAIR = 25
_RBF_TOT = _NPAIR * _NUM_RBF

_ATOM = {"N": 0, "Ca": 1, "C": 2, "O": 3, "Cb": 4}
_PAIRS = [("Ca", "Ca"),
          ("N", "N"), ("C", "C"), ("O", "O"), ("Cb", "Cb"), ("Ca", "N"),
          ("Ca", "C"), ("Ca", "O"), ("Ca", "Cb"), ("N", "C"), ("N", "O"),
          ("N", "Cb"), ("Cb", "C"), ("Cb", "O"), ("O", "C"), ("N", "Ca"),
          ("C", "Ca"), ("O", "Ca"), ("Cb", "Ca"), ("C", "N"), ("O", "N"),
          ("Cb", "N"), ("C", "Cb"), ("O", "Cb"), ("C", "O")]


def _build_consts():
    sel = np.zeros((2 * _TW, 3 * _NPAIR), dtype=np.float32)
    for p, (ai, aj) in enumerate(_PAIRS):
        for c in range(3):
            sel[3 * _ATOM[ai] + c, 3 * p + c] = 1.0
            sel[_TW + 3 * _ATOM[aj] + c, 3 * p + c] = -1.0
    t = np.zeros((3 * _NPAIR, _RBF_TOT), dtype=np.float32)
    for p in range(_NPAIR):
        for c in range(3):
            for m in range(_NUM_RBF):
                t[3 * p + c, p * _NUM_RBF + m] = 1.0
    mu = np.linspace(2.0, 22.0, _NUM_RBF).astype(np.float32)
    mu = np.tile(mu, _NPAIR).reshape(1, _RBF_TOT)
    return jnp.asarray(sel), jnp.asarray(t), jnp.asarray(mu)


def _topk_body(xr_ref, xt_ref, rid_ref, ch_ref, tab_ref, eidx_ref, d_ref):
    xr = xr_ref[0]
    n_a = xr[:, 0:3]
    ca = xr[:, 3:6]
    c_a = xr[:, 6:9]
    b_v = ca - n_a
    c_v = c_a - ca
    bx, by, bz = b_v[:, 0:1], b_v[:, 1:2], b_v[:, 2:3]
    cx, cy, cz = c_v[:, 0:1], c_v[:, 1:2], c_v[:, 2:3]
    a_v = jnp.concatenate(
        [by * cz - bz * cy, bz * cx - bx * cz, bx * cy - by * cx], axis=1)
    cb = -0.58273431 * a_v + 0.56802827 * b_v - 0.54067466 * c_v + ca

    rid = rid_ref[0]
    ch = ch_ref[0]
    pad = jnp.zeros((_L, _TW - 17), dtype=jnp.float32)
    tab_ref[0] = jnp.concatenate(
        [xr[:, 0:3], ca, xr[:, 6:12], cb, rid, ch, pad], axis=1)

    xt = xt_ref[0]
    d2 = jnp.zeros((_L, _L), dtype=jnp.float32)
    for c in range(3):
        dx = ca[:, c:c + 1] - xt[3 + c:4 + c, :]
        d2 = d2 + dx * dx
    d_ref[...] = jnp.sqrt(d2 + 1e-6)

    lane = lax.broadcasted_iota(jnp.int32, (_L, _L), 1)
    big = jnp.int32(1 << 20)
    for k in range(_K):
        d = d_ref[...]
        m = jnp.min(d, axis=1, keepdims=True)
        am = jnp.min(jnp.where(d == m, lane, big), axis=1, keepdims=True)
        eidx_ref[0, :, k:k + 1] = am
        d_ref[...] = jnp.where(lane == am, jnp.float32(jnp.inf), d)


def _stage_a(xr, xt, ridf, chf):
    return pl.pallas_call(
        _topk_body,
        grid=(_B,),
        in_specs=[
            pl.BlockSpec((1, _L, 12), lambda b: (b, 0, 0)),
            pl.BlockSpec((1, 12, _L), lambda b: (b, 0, 0)),
            pl.BlockSpec((1, _L, 1), lambda b: (b, 0, 0)),
            pl.BlockSpec((1, _L, 1), lambda b: (b, 0, 0)),
        ],
        out_specs=[
            pl.BlockSpec((1, _L, _TW), lambda b: (b, 0, 0)),
            pl.BlockSpec((1, _L, _K), lambda b: (b, 0, 0)),
        ],
        out_shape=[
            jax.ShapeDtypeStruct((_B, _L, _TW), jnp.float32),
            jax.ShapeDtypeStruct((_B, _L, _K), jnp.int32),
        ],
        scratch_shapes=[pltpu.VMEM((_L, _L), jnp.float32)],
    )(xr, xt, ridf, chf)


_NE = _B * _L * _K
_IDX_CH = 120
_N_CH = 8


def _sc_gather(tab_flat, gidx_j):
    info = plsc.get_sparse_core_info()
    nw = info.num_cores * info.num_subcores
    per_w = _NE // nw
    assert per_w == _IDX_CH * _N_CH

    mesh = plsc.VectorSubcoreMesh(core_axis_name="c", subcore_axis_name="s")

    @functools.partial(
        pl.kernel,
        out_type=jax.ShapeDtypeStruct((_NE, _TW), jnp.float32),
        mesh=mesh,
        scratch_types=[
            pltpu.VMEM((per_w,), jnp.int32),
            pltpu.VMEM((per_w, _TW), jnp.float32),
            pltpu.SemaphoreType.DMA,
        ],
    )
    def gather_k(tab_hbm, gj_hbm, oj_hbm, ij_v, rj_v, sem):
        wid = lax.axis_index("s") * info.num_cores + lax.axis_index("c")
        base = wid * per_w
        pltpu.sync_copy(gj_hbm.at[pl.ds(base, per_w)], ij_v)
        for c in range(_N_CH):
            sl = pl.ds(c * _IDX_CH, _IDX_CH)
            pltpu.async_copy(tab_hbm.at[ij_v.at[sl]], rj_v.at[sl], sem).wait()
        pltpu.sync_copy(rj_v, oj_hbm.at[pl.ds(base, per_w)])

    return gather_k(tab_flat, gidx_j)


_BLK = 960
_RPB = _BLK // _K


def _split(x):
    hi = x.astype(jnp.bfloat16)
    lo = (x - hi.astype(jnp.float32)).astype(jnp.bfloat16)
    return hi, lo


def _bdot(a, b):
    return jnp.dot(a, b, preferred_element_type=jnp.float32)


def _dot2(a, b):
    ah, al = _split(a)
    bh, bl = _split(b)
    return _bdot(ah, bh) + _bdot(al, bh) + _bdot(ah, bl)


def _feat_body(tab_ref, gj_ref, sel_ref, t_ref, mu_ref, pew_ref, peb_ref,
               eew_ref, lng_ref, lnb_ref, out_ref):
    row_of_edge = lax.broadcasted_iota(jnp.int32, (_BLK, _RPB), 0) // _K
    col = lax.broadcasted_iota(jnp.int32, (_BLK, _RPB), 1)
    rep = (row_of_edge == col).astype(jnp.bfloat16)
    tab_hi, tab_lo = _split(tab_ref[...])
    gi_hi = _bdot(rep, tab_hi)
    gi_lo = _bdot(rep, tab_lo)
    gi = gi_hi + gi_lo
    gj = gj_ref[...]
    gj_hi, gj_lo = _split(gj)
    sel = sel_ref[...].astype(jnp.bfloat16)
    gij_hi = jnp.concatenate([gi_hi.astype(jnp.bfloat16), gj_hi], axis=1)
    gij_lo = jnp.concatenate([gi_lo.astype(jnp.bfloat16), gj_lo], axis=1)
    dif = _bdot(gij_hi, sel) + _bdot(gij_lo, sel)
    sq = dif * dif
    sq_hi, sq_lo = _split(sq)
    t_b = t_ref[...].astype(jnp.bfloat16)
    d2 = _bdot(sq_hi, t_b) + _bdot(sq_lo, t_b)
    dist = jnp.sqrt(d2 + 1e-6)
    z = (dist - mu_ref[...]) * jnp.float32(_NUM_RBF / 20.0)
    rbf = jnp.exp(-(z * z))

    rid_i = gi[:, 15:16]
    rid_j = gj[:, 15:16]
    same = (gi[:, 16:17] == gj[:, 16:17]).astype(jnp.float32)
    off = jnp.clip(rid_i - rid_j + _MAXREL, 0.0, 2.0 * _MAXREL)
    d_pe = off * same + (1.0 - same) * (2.0 * _MAXREL + 1.0)
    lane = lax.broadcasted_iota(jnp.int32, (_BLK, 128), 1)
    onehot = (lane == d_pe.astype(jnp.int32)).astype(jnp.bfloat16)
    pew_hi, pew_lo = _split(pew_ref[...])
    e_pos = _bdot(onehot, pew_hi) + _bdot(onehot, pew_lo) + peb_ref[...]

    e = (_dot2(e_pos, eew_ref[0:_NUM_PE, :])
         + _dot2(rbf, eew_ref[_NUM_PE:, :]))
    mu_e = jnp.mean(e, axis=1, keepdims=True)
    ec = e - mu_e
    var = jnp.mean(ec * ec, axis=1, keepdims=True)
    out_ref[...] = ec * jax.lax.rsqrt(var + 1e-5) * lng_ref[...] \
        + lnb_ref[...]


def _stage_c(tab_flat, gj, sel, t, mu, pew_pad, peb, eew, lng, lnb):
    nprog = _NE // _BLK
    return pl.pallas_call(
        _feat_body,
        grid=(nprog,),
        in_specs=[
            pl.BlockSpec((_RPB, _TW), lambda i: (i, 0)),
            pl.BlockSpec((_BLK, _TW), lambda i: (i, 0)),
            pl.BlockSpec((2 * _TW, 3 * _NPAIR), lambda i: (0, 0)),
            pl.BlockSpec((3 * _NPAIR, _RBF_TOT), lambda i: (0, 0)),
            pl.BlockSpec((1, _RBF_TOT), lambda i: (0, 0)),
            pl.BlockSpec((128, _NUM_PE), lambda i: (0, 0)),
            pl.BlockSpec((1, _NUM_PE), lambda i: (0, 0)),
            pl.BlockSpec((_NUM_PE + _RBF_TOT, _EDGE_F), lambda i: (0, 0)),
            pl.BlockSpec((1, _EDGE_F), lambda i: (0, 0)),
            pl.BlockSpec((1, _EDGE_F), lambda i: (0, 0)),
        ],
        out_specs=pl.BlockSpec((_BLK, _EDGE_F), lambda i: (i, 0)),
        out_shape=jax.ShapeDtypeStruct((_NE, _EDGE_F), jnp.float32),
    )(tab_flat, gj, sel, t, mu, pew_pad, peb, eew, lng, lnb)


def kernel(X, X_m, Y, Y_m, Z, Z_m, Z_t, mask, residue_idx,
           chain_encoding_all, pe_w, pe_b, ee_w, ln_g, ln_b):
    xr = X.reshape(_B, _L, 12)
    xt = jnp.swapaxes(xr, 1, 2)
    ridf = residue_idx.astype(jnp.float32).reshape(_B, _L, 1)
    chf = chain_encoding_all.astype(jnp.float32).reshape(_B, _L, 1)

    table, e_idx = _stage_a(xr, xt, ridf, chf)

    gidx_j = (e_idx + (jnp.arange(_B, dtype=jnp.int32) * _L)[:, None, None]
              ).reshape(_NE)
    tab_flat = table.reshape(_B * _L, _TW)

    gj = _sc_gather(tab_flat, gidx_j)

    sel, t, mu = _build_consts()
    pew_pad = jnp.zeros((128, _NUM_PE), jnp.float32).at[0:66].set(pe_w)
    e = _stage_c(tab_flat, gj, sel, t, mu, pew_pad, pe_b.reshape(1, _NUM_PE),
                 ee_w, ln_g.reshape(1, _EDGE_F), ln_b.reshape(1, _EDGE_F))
    return e.reshape(_B, _L, _K, _EDGE_F), e_idx


# speedup vs baseline: 2.6163x; 1.1211x over previous
import functools

import jax
import jax.numpy as jnp
import numpy as np
from jax import lax
from jax.experimental import pallas as pl
from jax.experimental.pallas import tpu as pltpu
from jax.experimental.pallas import tpu_sc as plsc

_B, _L = 2, 512
_K = 30
_NUM_RBF = 16
_NUM_PE = 16
_EDGE_F = 128
_MAXREL = 32
_TW = 128
_GW = _TW
_NPAIR = 25
_RBF_TOT = _NPAIR * _NUM_RBF

_ATOM = {"N": 0, "Ca": 1, "C": 2, "O": 3, "Cb": 4}
_PAIRS = [("Ca", "Ca"),
          ("N", "N"), ("C", "C"), ("O", "O"), ("Cb", "Cb"), ("Ca", "N"),
          ("Ca", "C"), ("Ca", "O"), ("Ca", "Cb"), ("N", "C"), ("N", "O"),
          ("N", "Cb"), ("Cb", "C"), ("Cb", "O"), ("O", "C"), ("N", "Ca"),
          ("C", "Ca"), ("O", "Ca"), ("Cb", "Ca"), ("C", "N"), ("O", "N"),
          ("Cb", "N"), ("C", "Cb"), ("O", "Cb"), ("C", "O")]


def _build_consts():
    sel = np.zeros((_TW + _GW, 3 * _NPAIR), dtype=np.float32)
    for p, (ai, aj) in enumerate(_PAIRS):
        for c in range(3):
            sel[3 * _ATOM[ai] + c, 3 * p + c] = 1.0
            sel[_TW + 3 * _ATOM[aj] + c, 3 * p + c] = -1.0
    t = np.zeros((3 * _NPAIR, _RBF_TOT), dtype=np.float32)
    for p in range(_NPAIR):
        for c in range(3):
            for m in range(_NUM_RBF):
                t[3 * p + c, p * _NUM_RBF + m] = 1.0
    mu = np.linspace(2.0, 22.0, _NUM_RBF).astype(np.float32)
    mu = np.tile(mu, _NPAIR).reshape(1, _RBF_TOT)
    return jnp.asarray(sel), jnp.asarray(t), jnp.asarray(mu)


def _fold_pos_table(pe_w, pe_b, ee_w):
    p = jnp.dot(pe_w + pe_b[None, :], ee_w[0:_NUM_PE, :],
                precision=lax.Precision.HIGHEST)
    return jnp.zeros((128, _EDGE_F), jnp.float32).at[0:2 * _MAXREL + 2].set(p)


def _topk_body(xr_ref, xt_ref, rid_ref, ch_ref, tab_ref, eidx_ref, d_ref):
    xr = xr_ref[0]
    n_a = xr[:, 0:3]
    ca = xr[:, 3:6]
    c_a = xr[:, 6:9]
    b_v = ca - n_a
    c_v = c_a - ca
    bx, by, bz = b_v[:, 0:1], b_v[:, 1:2], b_v[:, 2:3]
    cx, cy, cz = c_v[:, 0:1], c_v[:, 1:2], c_v[:, 2:3]
    a_v = jnp.concatenate(
        [by * cz - bz * cy, bz * cx - bx * cz, bx * cy - by * cx], axis=1)
    cb = -0.58273431 * a_v + 0.56802827 * b_v - 0.54067466 * c_v + ca

    rid = rid_ref[0]
    ch = ch_ref[0]
    pad = jnp.zeros((_L, _TW - 17), dtype=jnp.float32)
    tab_ref[0] = jnp.concatenate(
        [xr[:, 0:3], ca, xr[:, 6:12], cb, rid, ch, pad], axis=1)

    xt = xt_ref[0]
    d2 = jnp.zeros((_L, _L), dtype=jnp.float32)
    for c in range(3):
        dx = ca[:, c:c + 1] - xt[3 + c:4 + c, :]
        d2 = d2 + dx * dx
    d_ref[...] = jnp.sqrt(d2 + 1e-6)

    lane = lax.broadcasted_iota(jnp.int32, (_L, _L), 1)
    big = jnp.int32(1 << 20)
    for k in range(_K):
        d = d_ref[...]
        m = jnp.min(d, axis=1, keepdims=True)
        am = jnp.min(jnp.where(d == m, lane, big), axis=1, keepdims=True)
        eidx_ref[0, :, k:k + 1] = am
        d_ref[...] = jnp.where(lane == am, jnp.float32(jnp.inf), d)


def _stage_a(xr, xt, ridf, chf):
    return pl.pallas_call(
        _topk_body,
        grid=(_B,),
        in_specs=[
            pl.BlockSpec((1, _L, 12), lambda b: (b, 0, 0)),
            pl.BlockSpec((1, 12, _L), lambda b: (b, 0, 0)),
            pl.BlockSpec((1, _L, 1), lambda b: (b, 0, 0)),
            pl.BlockSpec((1, _L, 1), lambda b: (b, 0, 0)),
        ],
        out_specs=[
            pl.BlockSpec((1, _L, _TW), lambda b: (b, 0, 0)),
            pl.BlockSpec((1, _L, _K), lambda b: (b, 0, 0)),
        ],
        out_shape=[
            jax.ShapeDtypeStruct((_B, _L, _TW), jnp.float32),
            jax.ShapeDtypeStruct((_B, _L, _K), jnp.int32),
        ],
        scratch_shapes=[pltpu.VMEM((_L, _L), jnp.float32)],
    )(xr, xt, ridf, chf)


_NE = _B * _L * _K
_IDX_CH = 120
_N_CH = 8


def _sc_gather(tab_flat, gidx_j):
    info = plsc.get_sparse_core_info()
    nw = info.num_cores * info.num_subcores
    per_w = _NE // nw
    assert per_w == _IDX_CH * _N_CH

    mesh = plsc.VectorSubcoreMesh(core_axis_name="c", subcore_axis_name="s")

    @functools.partial(
        pl.kernel,
        out_type=jax.ShapeDtypeStruct((_NE, _GW), jnp.float32),
        mesh=mesh,
        scratch_types=[
            pltpu.VMEM((per_w,), jnp.int32),
            pltpu.VMEM((per_w, _TW), jnp.float32),
            pltpu.SemaphoreType.DMA,
        ],
    )
    def gather_k(tab_hbm, gj_hbm, oj_hbm, ij_v, rj_v, sem):
        wid = lax.axis_index("s") * info.num_cores + lax.axis_index("c")
        base = wid * per_w
        pltpu.sync_copy(gj_hbm.at[pl.ds(base, per_w)], ij_v)
        for c in range(_N_CH):
            sl = pl.ds(c * _IDX_CH, _IDX_CH)
            pltpu.async_copy(tab_hbm.at[ij_v.at[sl]], rj_v.at[sl], sem).wait()
        pltpu.sync_copy(rj_v, oj_hbm.at[pl.ds(base, per_w)])

    return gather_k(tab_flat, gidx_j)


_BLK = 1920
_RPB = _BLK // _K


def _split(x):
    hi = x.astype(jnp.bfloat16)
    lo = (x - hi.astype(jnp.float32)).astype(jnp.bfloat16)
    return hi, lo


def _bdot(a, b):
    return jnp.dot(a, b, preferred_element_type=jnp.float32)


def _dot2(a, b):
    ah, al = _split(a)
    bh, bl = _split(b)
    return _bdot(ah, bh) + _bdot(al, bh) + _bdot(ah, bl)


def _feat_body(tab_ref, gj_ref, sel_ref, t_ref, mu_ref, posh_ref, posl_ref,
               eewh_ref, eewl_ref, lng_ref, lnb_ref, out_ref):
    row_of_edge = lax.broadcasted_iota(jnp.int32, (_BLK, _RPB), 0) // _K
    col = lax.broadcasted_iota(jnp.int32, (_BLK, _RPB), 1)
    rep = (row_of_edge == col).astype(jnp.bfloat16)
    tab_hi, tab_lo = _split(tab_ref[...])
    gi_hi = _bdot(rep, tab_hi)
    gi_lo = _bdot(rep, tab_lo)
    gi = gi_hi + gi_lo
    gj = gj_ref[...]
    gj_hi, gj_lo = _split(gj)
    sel = sel_ref[...].astype(jnp.bfloat16)
    gij_hi = jnp.concatenate([gi_hi.astype(jnp.bfloat16), gj_hi], axis=1)
    gij_lo = jnp.concatenate([gi_lo.astype(jnp.bfloat16), gj_lo], axis=1)
    dif = _bdot(gij_hi, sel) + _bdot(gij_lo, sel)
    sq = dif * dif
    sq_hi, sq_lo = _split(sq)
    t_b = t_ref[...].astype(jnp.bfloat16)
    d2 = _bdot(sq_hi, t_b) + _bdot(sq_lo, t_b)
    dist = jnp.sqrt(d2 + 1e-6)
    z = (dist - mu_ref[...]) * jnp.float32(_NUM_RBF / 20.0)
    rbf = jnp.exp(-(z * z))

    rid_i = gi[:, 15:16]
    rid_j = gj[:, 15:16]
    same = (gi[:, 16:17] == gj[:, 16:17]).astype(jnp.float32)
    off = jnp.clip(rid_i - rid_j + _MAXREL, 0.0, 2.0 * _MAXREL)
    d_pe = off * same + (1.0 - same) * (2.0 * _MAXREL + 1.0)
    lane = lax.broadcasted_iota(jnp.int32, (_BLK, 128), 1)
    onehot = (lane == d_pe.astype(jnp.int32)).astype(jnp.bfloat16)
    e_posc = _bdot(onehot, posh_ref[...]) + _bdot(onehot, posl_ref[...])

    rbf_hi, rbf_lo = _split(rbf)
    e = (e_posc + _bdot(rbf_hi, eewh_ref[...]) + _bdot(rbf_lo, eewh_ref[...])
         + _bdot(rbf_hi, eewl_ref[...]))
    mu_e = jnp.mean(e, axis=1, keepdims=True)
    ec = e - mu_e
    var = jnp.mean(ec * ec, axis=1, keepdims=True)
    out_ref[...] = ec * jax.lax.rsqrt(var + 1e-5) * lng_ref[...] \
        + lnb_ref[...]


def _stage_c(tab_flat, gj, sel, t, mu, posh, posl, eewh, eewl, lng, lnb):
    nprog = _NE // _BLK
    return pl.pallas_call(
        _feat_body,
        grid=(nprog,),
        in_specs=[
            pl.BlockSpec((_RPB, _TW), lambda i: (i, 0)),
            pl.BlockSpec((_BLK, _GW), lambda i: (i, 0)),
            pl.BlockSpec((_TW + _GW, 3 * _NPAIR), lambda i: (0, 0)),
            pl.BlockSpec((3 * _NPAIR, _RBF_TOT), lambda i: (0, 0)),
            pl.BlockSpec((1, _RBF_TOT), lambda i: (0, 0)),
            pl.BlockSpec((128, _EDGE_F), lambda i: (0, 0)),
            pl.BlockSpec((128, _EDGE_F), lambda i: (0, 0)),
            pl.BlockSpec((_RBF_TOT, _EDGE_F), lambda i: (0, 0)),
            pl.BlockSpec((_RBF_TOT, _EDGE_F), lambda i: (0, 0)),
            pl.BlockSpec((1, _EDGE_F), lambda i: (0, 0)),
            pl.BlockSpec((1, _EDGE_F), lambda i: (0, 0)),
        ],
        out_specs=pl.BlockSpec((_BLK, _EDGE_F), lambda i: (i, 0)),
        out_shape=jax.ShapeDtypeStruct((_NE, _EDGE_F), jnp.float32),
    )(tab_flat, gj, sel, t, mu, posh, posl, eewh, eewl, lng, lnb)


def kernel(X, X_m, Y, Y_m, Z, Z_m, Z_t, mask, residue_idx,
           chain_encoding_all, pe_w, pe_b, ee_w, ln_g, ln_b):
    xr = X.reshape(_B, _L, 12)
    xt = jnp.swapaxes(xr, 1, 2)
    ridf = residue_idx.astype(jnp.float32).reshape(_B, _L, 1)
    chf = chain_encoding_all.astype(jnp.float32).reshape(_B, _L, 1)

    table, e_idx = _stage_a(xr, xt, ridf, chf)

    gidx_j = (e_idx + (jnp.arange(_B, dtype=jnp.int32) * _L)[:, None, None]
              ).reshape(_NE)
    tab_flat = table.reshape(_B * _L, _TW)

    gj = _sc_gather(tab_flat, gidx_j)

    sel, t, mu = _build_consts()
    pos_tab = _fold_pos_table(pe_w, pe_b, ee_w)
    posh, posl = _split(pos_tab)
    eewh, eewl = _split(ee_w[_NUM_PE:, :])
    e = _stage_c(tab_flat, gj, sel, t, mu, posh, posl, eewh, eewl,
                 ln_g.reshape(1, _EDGE_F), ln_b.reshape(1, _EDGE_F))
    return e.reshape(_B, _L, _K, _EDGE_F), e_idx


# speedup vs baseline: 2.6858x; 1.0266x over previous
import functools

import jax
import jax.numpy as jnp
import numpy as np
from jax import lax
from jax.experimental import pallas as pl
from jax.experimental.pallas import tpu as pltpu
from jax.experimental.pallas import tpu_sc as plsc

_B, _L = 2, 512
_K = 30
_NUM_RBF = 16
_NUM_PE = 16
_EDGE_F = 128
_MAXREL = 32
_TW = 128
_GW = _TW
_NPAIR = 25
_RBF_TOT = _NPAIR * _NUM_RBF

_ATOM = {"N": 0, "Ca": 1, "C": 2, "O": 3, "Cb": 4}
_PAIRS = [("Ca", "Ca"),
          ("N", "N"), ("C", "C"), ("O", "O"), ("Cb", "Cb"), ("Ca", "N"),
          ("Ca", "C"), ("Ca", "O"), ("Ca", "Cb"), ("N", "C"), ("N", "O"),
          ("N", "Cb"), ("Cb", "C"), ("Cb", "O"), ("O", "C"), ("N", "Ca"),
          ("C", "Ca"), ("O", "Ca"), ("Cb", "Ca"), ("C", "N"), ("O", "N"),
          ("Cb", "N"), ("C", "Cb"), ("O", "Cb"), ("C", "O")]


def _build_consts():
    sel = np.zeros((_TW + _GW, 3 * _NPAIR), dtype=np.float32)
    for p, (ai, aj) in enumerate(_PAIRS):
        for c in range(3):
            sel[3 * _ATOM[ai] + c, 3 * p + c] = 1.0
            sel[_TW + 3 * _ATOM[aj] + c, 3 * p + c] = -1.0
    t = np.zeros((3 * _NPAIR, _NPAIR), dtype=np.float32)
    r = np.zeros((_NPAIR, _RBF_TOT), dtype=np.float32)
    for p in range(_NPAIR):
        for c in range(3):
            t[3 * p + c, p] = 1.0
        for m in range(_NUM_RBF):
            r[p, p * _NUM_RBF + m] = 1.0
    mu = np.linspace(2.0, 22.0, _NUM_RBF).astype(np.float32)
    mu = np.tile(mu, _NPAIR).reshape(1, _RBF_TOT)
    return jnp.asarray(sel), jnp.asarray(t), jnp.asarray(r), jnp.asarray(mu)


def _fold_pos_table(pe_w, pe_b, ee_w):
    p = jnp.dot(pe_w + pe_b[None, :], ee_w[0:_NUM_PE, :],
                precision=lax.Precision.HIGHEST)
    return jnp.zeros((128, _EDGE_F), jnp.float32).at[0:2 * _MAXREL + 2].set(p)


def _topk_body(xr_ref, xt_ref, rid_ref, ch_ref, tab_ref, eidx_ref, d_ref):
    xr = xr_ref[0]
    n_a = xr[:, 0:3]
    ca = xr[:, 3:6]
    c_a = xr[:, 6:9]
    b_v = ca - n_a
    c_v = c_a - ca
    bx, by, bz = b_v[:, 0:1], b_v[:, 1:2], b_v[:, 2:3]
    cx, cy, cz = c_v[:, 0:1], c_v[:, 1:2], c_v[:, 2:3]
    a_v = jnp.concatenate(
        [by * cz - bz * cy, bz * cx - bx * cz, bx * cy - by * cx], axis=1)
    cb = -0.58273431 * a_v + 0.56802827 * b_v - 0.54067466 * c_v + ca

    rid = rid_ref[0]
    ch = ch_ref[0]
    pad = jnp.zeros((_L, _TW - 17), dtype=jnp.float32)
    tab_ref[0] = jnp.concatenate(
        [xr[:, 0:3], ca, xr[:, 6:12], cb, rid, ch, pad], axis=1)

    xt = xt_ref[0]
    d2 = jnp.zeros((_L, _L), dtype=jnp.float32)
    for c in range(3):
        dx = ca[:, c:c + 1] - xt[3 + c:4 + c, :]
        d2 = d2 + dx * dx
    d_ref[...] = jnp.sqrt(d2 + 1e-6)

    lane = lax.broadcasted_iota(jnp.int32, (_L, _L), 1)
    big = jnp.int32(1 << 20)
    for k in range(_K):
        d = d_ref[...]
        m = jnp.min(d, axis=1, keepdims=True)
        am = jnp.min(jnp.where(d == m, lane, big), axis=1, keepdims=True)
        eidx_ref[0, :, k:k + 1] = am
        d_ref[...] = jnp.where(lane == am, jnp.float32(jnp.inf), d)


def _stage_a(xr, xt, ridf, chf):
    return pl.pallas_call(
        _topk_body,
        grid=(_B,),
        in_specs=[
            pl.BlockSpec((1, _L, 12), lambda b: (b, 0, 0)),
            pl.BlockSpec((1, 12, _L), lambda b: (b, 0, 0)),
            pl.BlockSpec((1, _L, 1), lambda b: (b, 0, 0)),
            pl.BlockSpec((1, _L, 1), lambda b: (b, 0, 0)),
        ],
        out_specs=[
            pl.BlockSpec((1, _L, _TW), lambda b: (b, 0, 0)),
            pl.BlockSpec((1, _L, _K), lambda b: (b, 0, 0)),
        ],
        out_shape=[
            jax.ShapeDtypeStruct((_B, _L, _TW), jnp.float32),
            jax.ShapeDtypeStruct((_B, _L, _K), jnp.int32),
        ],
        scratch_shapes=[pltpu.VMEM((_L, _L), jnp.float32)],
    )(xr, xt, ridf, chf)


_NE = _B * _L * _K
_IDX_CH = 120
_N_CH = 8


def _sc_gather(tab_flat, gidx_j):
    info = plsc.get_sparse_core_info()
    nw = info.num_cores * info.num_subcores
    per_w = _NE // nw
    assert per_w == _IDX_CH * _N_CH

    mesh = plsc.VectorSubcoreMesh(core_axis_name="c", subcore_axis_name="s")

    @functools.partial(
        pl.kernel,
        out_type=jax.ShapeDtypeStruct((_NE, _GW), jnp.float32),
        mesh=mesh,
        scratch_types=[
            pltpu.VMEM((per_w,), jnp.int32),
            pltpu.VMEM((per_w, _TW), jnp.float32),
            pltpu.SemaphoreType.DMA,
        ],
    )
    def gather_k(tab_hbm, gj_hbm, oj_hbm, ij_v, rj_v, sem):
        wid = lax.axis_index("s") * info.num_cores + lax.axis_index("c")
        base = wid * per_w
        pltpu.sync_copy(gj_hbm.at[pl.ds(base, per_w)], ij_v)
        for c in range(_N_CH):
            sl = pl.ds(c * _IDX_CH, _IDX_CH)
            pltpu.async_copy(tab_hbm.at[ij_v.at[sl]], rj_v.at[sl], sem).wait()
        pltpu.sync_copy(rj_v, oj_hbm.at[pl.ds(base, per_w)])

    return gather_k(tab_flat, gidx_j)


_BLK = 1920
_RPB = _BLK // _K


def _split(x):
    hi = x.astype(jnp.bfloat16)
    lo = (x - hi.astype(jnp.float32)).astype(jnp.bfloat16)
    return hi, lo


def _bdot(a, b):
    return jnp.dot(a, b, preferred_element_type=jnp.float32)


def _dot2(a, b):
    ah, al = _split(a)
    bh, bl = _split(b)
    return _bdot(ah, bh) + _bdot(al, bh) + _bdot(ah, bl)


def _feat_body(tab_ref, gj_ref, sel_ref, t_ref, r_ref, mu_ref, posh_ref,
               posl_ref, eewh_ref, eewl_ref, lng_ref, lnb_ref, out_ref):
    row_of_edge = lax.broadcasted_iota(jnp.int32, (_BLK, _RPB), 0) // _K
    col = lax.broadcasted_iota(jnp.int32, (_BLK, _RPB), 1)
    rep = (row_of_edge == col).astype(jnp.bfloat16)
    tab_hi, tab_lo = _split(tab_ref[...])
    gi_hi = _bdot(rep, tab_hi)
    gi_lo = _bdot(rep, tab_lo)
    gi = gi_hi + gi_lo
    gj = gj_ref[...]
    gj_hi, gj_lo = _split(gj)
    sel = sel_ref[...].astype(jnp.bfloat16)
    gij_hi = jnp.concatenate([gi_hi.astype(jnp.bfloat16), gj_hi], axis=1)
    gij_lo = jnp.concatenate([gi_lo.astype(jnp.bfloat16), gj_lo], axis=1)
    dif = _bdot(gij_hi, sel) + _bdot(gij_lo, sel)
    sq = dif * dif
    sq_hi, sq_lo = _split(sq)
    t_b = t_ref[...].astype(jnp.bfloat16)
    d2_25 = _bdot(sq_hi, t_b) + _bdot(sq_lo, t_b)
    dist25 = jnp.sqrt(d2_25 + 1e-6)
    d25_hi, d25_lo = _split(dist25)
    r_b = r_ref[...].astype(jnp.bfloat16)
    dist = _bdot(d25_hi, r_b) + _bdot(d25_lo, r_b)
    z = (dist - mu_ref[...]) * jnp.float32(_NUM_RBF / 20.0)
    rbf = jnp.exp(-(z * z))

    rid_i = gi[:, 15:16]
    rid_j = gj[:, 15:16]
    same = (gi[:, 16:17] == gj[:, 16:17]).astype(jnp.float32)
    off = jnp.clip(rid_i - rid_j + _MAXREL, 0.0, 2.0 * _MAXREL)
    d_pe = off * same + (1.0 - same) * (2.0 * _MAXREL + 1.0)
    lane = lax.broadcasted_iota(jnp.int32, (_BLK, 128), 1)
    onehot = (lane == d_pe.astype(jnp.int32)).astype(jnp.bfloat16)
    e_posc = _bdot(onehot, posh_ref[...]) + _bdot(onehot, posl_ref[...])

    rbf_hi, rbf_lo = _split(rbf)
    e = (e_posc + _bdot(rbf_hi, eewh_ref[...]) + _bdot(rbf_lo, eewh_ref[...])
         + _bdot(rbf_hi, eewl_ref[...]))
    mu_e = jnp.mean(e, axis=1, keepdims=True)
    ec = e - mu_e
    var = jnp.mean(ec * ec, axis=1, keepdims=True)
    out_ref[...] = ec * jax.lax.rsqrt(var + 1e-5) * lng_ref[...] \
        + lnb_ref[...]


def _stage_c(tab_flat, gj, sel, t, r, mu, posh, posl, eewh, eewl, lng, lnb):
    nprog = _NE // _BLK
    return pl.pallas_call(
        _feat_body,
        grid=(nprog,),
        in_specs=[
            pl.BlockSpec((_RPB, _TW), lambda i: (i, 0)),
            pl.BlockSpec((_BLK, _GW), lambda i: (i, 0)),
            pl.BlockSpec((_TW + _GW, 3 * _NPAIR), lambda i: (0, 0)),
            pl.BlockSpec((3 * _NPAIR, _NPAIR), lambda i: (0, 0)),
            pl.BlockSpec((_NPAIR, _RBF_TOT), lambda i: (0, 0)),
            pl.BlockSpec((1, _RBF_TOT), lambda i: (0, 0)),
            pl.BlockSpec((128, _EDGE_F), lambda i: (0, 0)),
            pl.BlockSpec((128, _EDGE_F), lambda i: (0, 0)),
            pl.BlockSpec((_RBF_TOT, _EDGE_F), lambda i: (0, 0)),
            pl.BlockSpec((_RBF_TOT, _EDGE_F), lambda i: (0, 0)),
            pl.BlockSpec((1, _EDGE_F), lambda i: (0, 0)),
            pl.BlockSpec((1, _EDGE_F), lambda i: (0, 0)),
        ],
        out_specs=pl.BlockSpec((_BLK, _EDGE_F), lambda i: (i, 0)),
        out_shape=jax.ShapeDtypeStruct((_NE, _EDGE_F), jnp.float32),
    )(tab_flat, gj, sel, t, r, mu, posh, posl, eewh, eewl, lng, lnb)


def kernel(X, X_m, Y, Y_m, Z, Z_m, Z_t, mask, residue_idx,
           chain_encoding_all, pe_w, pe_b, ee_w, ln_g, ln_b):
    xr = X.reshape(_B, _L, 12)
    xt = jnp.swapaxes(xr, 1, 2)
    ridf = residue_idx.astype(jnp.float32).reshape(_B, _L, 1)
    chf = chain_encoding_all.astype(jnp.float32).reshape(_B, _L, 1)

    table, e_idx = _stage_a(xr, xt, ridf, chf)

    gidx_j = (e_idx + (jnp.arange(_B, dtype=jnp.int32) * _L)[:, None, None]
              ).reshape(_NE)
    tab_flat = table.reshape(_B * _L, _TW)

    gj = _sc_gather(tab_flat, gidx_j)

    sel, t, r, mu = _build_consts()
    pos_tab = _fold_pos_table(pe_w, pe_b, ee_w)
    posh, posl = _split(pos_tab)
    eewh, eewl = _split(ee_w[_NUM_PE:, :])
    e = _stage_c(tab_flat, gj, sel, t, r, mu, posh, posl, eewh, eewl,
                 ln_g.reshape(1, _EDGE_F), ln_b.reshape(1, _EDGE_F))
    return e.reshape(_B, _L, _K, _EDGE_F), e_idx
